# baseline (speedup 1.0000x reference)
import functools

import jax
import jax.numpy as jnp
from jax import lax
from jax.experimental import pallas as pl
from jax.experimental.pallas import tpu as pltpu
from jax.experimental.pallas import tpu_sc as plsc

NUM_CLOUDS = 4
PTS = 2500
M1 = 1250
M1P = 1280
M2 = 313
M2P = 320
NN = 64
R2_1 = 0.2 * 0.2
R2_2 = 0.4 * 0.4


def _fps_body(m, pk, P, px_ref, py_ref, pz_ref, out_ref):
    R = px_ref.shape[0]
    Pp = px_ref.shape[1]
    C = R // pk
    X = px_ref[...]
    Y = py_ref[...]
    Z = pz_ref[...]
    iota = (lax.broadcasted_iota(jnp.int32, (R, Pp), 1)
            + (lax.broadcasted_iota(jnp.int32, (R, Pp), 0) % pk) * Pp)

    def pick(sel3, A):
        v = jnp.sum(jnp.where(sel3, A, 0.0), axis=1, keepdims=True)
        return jnp.sum(v.reshape(C, pk, 1), axis=1)

    def rep(v):
        return jnp.broadcast_to(v[:, None, :], (C, pk, 1)).reshape(R, 1)

    def step(i, dist, sel):
        jx = pick(sel, X)
        jy = pick(sel, Y)
        jz = pick(sel, Z)
        out_ref[:, pl.ds(i, 1), :] = jnp.concatenate(
            [jx, jy, jz], axis=1).reshape(C, 1, 3)
        ddx = X - rep(jx)
        ddy = Y - rep(jy)
        ddz = Z - rep(jz)
        return jnp.minimum(dist, ddx * ddx + ddy * ddy + ddz * ddz)

    big = jnp.full((R, Pp), jnp.float32(jnp.inf))
    dist0 = step(0, big, iota == 0)

    def body(i, dist):
        mxr = jnp.max(dist.reshape(C, pk, Pp), axis=1)
        mx = jnp.max(mxr, axis=1, keepdims=True)
        cand = jnp.where(dist == rep(mx), iota, P)
        idr = jnp.min(cand.reshape(C, pk, Pp), axis=1)
        idx = jnp.min(idr, axis=1, keepdims=True)
        return step(i, dist, iota == rep(idx))

    lax.fori_loop(1, m, body, dist0)


def _fps(pos_c, m, pk):
    C, P, _ = pos_c.shape
    px = pos_c[:, :, 0].reshape(C * pk, P // pk)
    py = pos_c[:, :, 1].reshape(C * pk, P // pk)
    pz = pos_c[:, :, 2].reshape(C * pk, P // pk)
    return pl.pallas_call(
        functools.partial(_fps_body, m, pk, P),
        out_shape=jax.ShapeDtypeStruct((C, m, 3), jnp.float32),
    )(px, py, pz)


def _radius_body(r2, n, sx_ref, sy_ref, sz_ref, dx_ref, dy_ref, dz_ref,
                 col_ref, cnt_ref):
    cid = pl.program_id(0)
    sx = sx_ref[0]
    sy = sy_ref[0]
    sz = sz_ref[0]
    dx = dx_ref[0]
    dy = dy_ref[0]
    dz = dz_ref[0]
    q = dx.shape[0]
    tri = (lax.broadcasted_iota(jnp.int32, (128, 128), 0)
           <= lax.broadcasted_iota(jnp.int32, (128, 128), 1)
           ).astype(jnp.bfloat16)
    pieces = []
    prefix = jnp.zeros((q, 1), jnp.float32)
    a = 0
    while a < n:
        w = min(128, n - a)
        ddx = dx - sx[:, a:a + w]
        ddy = dy - sy[:, a:a + w]
        ddz = dz - sz[:, a:a + w]
        d2 = ddx * ddx + ddy * ddy + ddz * ddz
        wint = (d2 <= r2).astype(jnp.bfloat16)
        local = jnp.dot(wint, tri[:w, :w],
                        preferred_element_type=jnp.float32)
        ci_t = local + prefix
        prefix = ci_t[:, w - 1:w]
        pieces.append(ci_t)
        a += w
    ci = jnp.concatenate(pieces, axis=1)
    cnt = prefix.astype(jnp.int32)
    cols = []
    for k in range(NN):
        cols.append(jnp.sum((ci <= jnp.float32(k)).astype(jnp.int32),
                            axis=1, keepdims=True))
    col = jnp.concatenate(cols, axis=1)
    col = jnp.minimum(col, n - 1) + cid * n
    col_ref[0] = col
    cnt_ref[0] = cnt


def _radius(pos_src_c, pos_dst_c, r2):
    C, n, _ = pos_src_c.shape
    q = pos_dst_c.shape[1]
    srcs = [pos_src_c[:, :, d].reshape(C, 1, n) for d in range(3)]
    dsts = [pos_dst_c[:, :, d:d + 1] for d in range(3)]
    grid = (C,)
    in_specs = (
        [pl.BlockSpec((1, 1, n), lambda c: (c, 0, 0)) for _ in range(3)]
        + [pl.BlockSpec((1, q, 1), lambda c: (c, 0, 0)) for _ in range(3)]
    )
    out_specs = [
        pl.BlockSpec((1, q, NN), lambda c: (c, 0, 0)),
        pl.BlockSpec((1, q, 1), lambda c: (c, 0, 0)),
    ]
    return pl.pallas_call(
        functools.partial(_radius_body, r2, n),
        grid=grid,
        in_specs=in_specs,
        out_specs=out_specs,
        out_shape=[
            jax.ShapeDtypeStruct((C, q, NN), jnp.int32),
            jax.ShapeDtypeStruct((C, q, 1), jnp.int32),
        ],
    )(*srcs, *dsts)


UNROLL = 2


def _sc_gather(table, idx2d):
    V, D = table.shape
    n_rows, CH = idx2d.shape
    B = n_rows * CH
    info = plsc.get_sparse_core_info()
    NC, NS = info.num_cores, info.num_subcores
    NW = NC * NS
    n_ch = n_rows // NW
    n_g = n_ch // UNROLL
    mesh = plsc.VectorSubcoreMesh(core_axis_name="c", subcore_axis_name="s")

    @functools.partial(
        pl.kernel,
        mesh=mesh,
        compiler_params=pltpu.CompilerParams(use_tc_tiling_on_sc=False),
        out_type=jax.ShapeDtypeStruct((B, D), jnp.float32),
        scratch_types=(
            [pltpu.VMEM((n_ch, CH), jnp.int32)]
            + [pltpu.VMEM((CH, D), jnp.float32) for _ in range(UNROLL)]
            + [pltpu.SemaphoreType.DMA for _ in range(2 * UNROLL)]
        ),
    )
    def k(table_hbm, idx_hbm, out_hbm, idx_v, *bufs_sems):
        bufs = bufs_sems[:UNROLL]
        gsems = bufs_sems[UNROLL:2 * UNROLL]
        osems = bufs_sems[2 * UNROLL:]
        wid = lax.axis_index("s") * NC + lax.axis_index("c")
        base = wid * n_ch
        pltpu.sync_copy(idx_hbm.at[pl.ds(base, n_ch)], idx_v)

        def body(g, carry):
            j0 = g * UNROLL
            cps = [
                pltpu.async_copy(table_hbm.at[idx_v.at[j0 + u]], bufs[u],
                                 gsems[u])
                for u in range(UNROLL)
            ]
            outs = []
            for u in range(UNROLL):
                cps[u].wait()
                outs.append(pltpu.async_copy(
                    bufs[u], out_hbm.at[pl.ds((base + j0 + u) * CH, CH)],
                    osems[u]))
            for o in outs:
                o.wait()
            return carry

        lax.fori_loop(0, n_g, body, 0)

    return k(table, idx2d)


def _conv_body(x_dim, rows_ref, dpos_ref, cnt_ref, w0_ref, b0_ref, w1_ref,
               b1_ref, w2_ref, b2_ref, out_ref):
    rows = rows_ref[...]
    dpos = dpos_ref[0]
    cnt = cnt_ref[0]
    Bq = dpos.shape[0]
    Be = rows.shape[0]
    drep = jnp.broadcast_to(dpos[:, None, :], (Bq, NN, 3)).reshape(Be, 3)
    rel = (rows[:, x_dim:x_dim + 3] - drep).astype(jnp.bfloat16)
    w0 = w0_ref[...].astype(jnp.bfloat16)
    if x_dim > 0:
        z = (jnp.dot(rows[:, :x_dim].astype(jnp.bfloat16), w0[:x_dim],
                     preferred_element_type=jnp.float32)
             + jnp.dot(rel, w0[x_dim:x_dim + 3],
                       preferred_element_type=jnp.float32)) + b0_ref[...]
    else:
        z = jnp.dot(rel, w0[:3], preferred_element_type=jnp.float32) \
            + b0_ref[...]
    h = jnp.maximum(z, 0.0).astype(jnp.bfloat16)
    z = jnp.dot(h, w1_ref[...].astype(jnp.bfloat16),
                preferred_element_type=jnp.float32) + b1_ref[...]
    h = jnp.maximum(z, 0.0).astype(jnp.bfloat16)
    z = jnp.dot(h, w2_ref[...].astype(jnp.bfloat16),
                preferred_element_type=jnp.float32) + b2_ref[...]
    k_e = lax.broadcasted_iota(jnp.int32, (Bq, NN, 1), 1).reshape(Be, 1)
    cnt_rep = jnp.broadcast_to(cnt[:, None, :], (Bq, NN, 1)).reshape(Be, 1)
    elig = k_e < cnt_rep
    zm = jnp.where(elig, z, -jnp.inf)
    y = jnp.max(zm.reshape(Bq, NN, z.shape[1]), axis=1)
    y = jnp.where(cnt > 0, y, 0.0)
    out_ref[0] = y


def _conv(rows, dst_pos_c, cnt, params, x_dim, bq):
    C, q, _ = dst_pos_c.shape
    (w0, b0), (w1, b1), (w2, b2) = params
    dout = w2.shape[1]
    dt = rows.shape[1]
    nb = q // bq
    be = bq * NN
    grid = (C, nb)
    in_specs = [
        pl.BlockSpec((be, dt), lambda c, b: (c * nb + b, 0)),
        pl.BlockSpec((1, bq, 3), lambda c, b: (c, b, 0)),
        pl.BlockSpec((1, bq, 1), lambda c, b: (c, b, 0)),
    ]
    for wgt, bias in ((w0, b0), (w1, b1), (w2, b2)):
        in_specs.append(
            pl.BlockSpec(wgt.shape, lambda c, b: (0, 0)))
        in_specs.append(
            pl.BlockSpec((1, bias.shape[0]), lambda c, b: (0, 0)))
    return pl.pallas_call(
        functools.partial(_conv_body, x_dim),
        grid=grid,
        in_specs=in_specs,
        out_specs=pl.BlockSpec((1, bq, dout), lambda c, b: (c, b, 0)),
        out_shape=jax.ShapeDtypeStruct((C, q, dout), jnp.float32),
    )(rows, dst_pos_c, cnt,
      w0, b0.reshape(1, -1), w1, b1.reshape(1, -1), w2, b2.reshape(1, -1))


def _sa3_body(x_ref, p_ref, w0_ref, b0_ref, w1_ref, b1_ref, w2_ref, b2_ref,
              out_ref):
    x = x_ref[0].astype(jnp.bfloat16)
    p = p_ref[0].astype(jnp.bfloat16)
    w0 = w0_ref[...].astype(jnp.bfloat16)
    z = (jnp.dot(x, w0[:256], preferred_element_type=jnp.float32)
         + jnp.dot(p, w0[256:259], preferred_element_type=jnp.float32)) \
        + b0_ref[...]
    h = jnp.maximum(z, 0.0).astype(jnp.bfloat16)
    z = jnp.dot(h, w1_ref[...].astype(jnp.bfloat16),
                preferred_element_type=jnp.float32) + b1_ref[...]
    h = jnp.maximum(z, 0.0).astype(jnp.bfloat16)
    z = jnp.dot(h, w2_ref[...].astype(jnp.bfloat16),
                preferred_element_type=jnp.float32) + b2_ref[...]
    out_ref[0, 0, :] = jnp.max(z, axis=0)


def _sa3(x2_c, pos2_c, params):
    C, m, _ = x2_c.shape
    (w0, b0), (w1, b1), (w2, b2) = params
    enc = w2.shape[1]
    in_specs = [
        pl.BlockSpec((1, m, 256), lambda c: (c, 0, 0)),
        pl.BlockSpec((1, m, 3), lambda c: (c, 0, 0)),
    ]
    for wgt, bias in ((w0, b0), (w1, b1), (w2, b2)):
        in_specs.append(pl.BlockSpec(wgt.shape, lambda c: (0, 0)))
        in_specs.append(pl.BlockSpec((1, bias.shape[0]), lambda c: (0, 0)))
    out = pl.pallas_call(
        _sa3_body,
        grid=(C,),
        in_specs=in_specs,
        out_specs=pl.BlockSpec((1, 1, enc), lambda c: (c, 0, 0)),
        out_shape=jax.ShapeDtypeStruct((C, 1, enc), jnp.float32),
    )(x2_c, pos2_c,
      w0, b0.reshape(1, -1), w1, b1.reshape(1, -1), w2, b2.reshape(1, -1))
    return out.reshape(C, enc)


def kernel(pos, ptr, mlp1, mlp2, mlp3):
    C = ptr.shape[0] - 1
    P = pos.shape[0] // C
    pos_c = pos.reshape(C, P, 3)

    pos1_c = _fps(pos_c, M1, 4)
    pos1p_c = jnp.concatenate(
        [pos1_c, jnp.full((C, M1P - M1, 3), 1e6, jnp.float32)], axis=1)
    col1, cnt1 = _radius(pos_c, pos1p_c, R2_1)
    e1 = C * M1P * NN
    idx1 = col1.reshape(e1 // 256, 256)
    table1 = jnp.concatenate(
        [pos, jnp.zeros((C * P, 13), jnp.float32)], axis=1)
    rows1 = _sc_gather(table1, idx1)
    x1_c = _conv(rows1, pos1p_c, cnt1, mlp1, 0, 128)

    pos2_c = _fps(pos1_c, M2, 2)
    pos2p_c = jnp.concatenate(
        [pos2_c, jnp.full((C, M2P - M2, 3), 1e6, jnp.float32)], axis=1)
    col2, cnt2 = _radius(pos1_c, pos2p_c, R2_2)
    e2 = C * M2P * NN
    idx2 = col2.reshape(e2 // 256, 256)
    table2 = jnp.concatenate(
        [x1_c[:, :M1].reshape(C * M1, 128), pos1_c.reshape(C * M1, 3),
         jnp.zeros((C * M1, 13), jnp.float32)], axis=1)
    rows2 = _sc_gather(table2, idx2)
    x2_c = _conv(rows2, pos2p_c, cnt2, mlp2, 128, 160)

    return _sa3(x2_c[:, :M2], pos2_c, mlp3)


# speedup vs baseline: 1.0454x; 1.0454x over previous
import functools

import jax
import jax.numpy as jnp
from jax import lax
from jax.experimental import pallas as pl
from jax.experimental.pallas import tpu as pltpu
from jax.experimental.pallas import tpu_sc as plsc

NUM_CLOUDS = 4
PTS = 2500
M1 = 1250
M1P = 1280
M2 = 313
M2P = 320
NN = 64
R2_1 = 0.2 * 0.2
R2_2 = 0.4 * 0.4


def _fps_body(m, px_ref, py_ref, pz_ref, out_ref):
    C = px_ref.shape[0]
    P = px_ref.shape[1]
    X = px_ref[...]
    Y = py_ref[...]
    Z = pz_ref[...]
    iota = lax.broadcasted_iota(jnp.int32, (C, P), 1)
    x0 = X[:, 0:1]
    y0 = Y[:, 0:1]
    z0 = Z[:, 0:1]
    out_ref[:, pl.ds(0, 1), :] = jnp.concatenate(
        [x0, y0, z0], axis=1).reshape(C, 1, 3)
    dx = X - x0
    dy = Y - y0
    dz = Z - z0
    dist0 = dx * dx + dy * dy + dz * dz

    def body(i, dist):
        mx = jnp.max(dist, axis=1, keepdims=True)
        idx = jnp.min(jnp.where(dist == mx, iota, P), axis=1, keepdims=True)
        sel = iota == idx
        jx = jnp.sum(jnp.where(sel, X, 0.0), axis=1, keepdims=True)
        jy = jnp.sum(jnp.where(sel, Y, 0.0), axis=1, keepdims=True)
        jz = jnp.sum(jnp.where(sel, Z, 0.0), axis=1, keepdims=True)
        out_ref[:, pl.ds(i, 1), :] = jnp.concatenate(
            [jx, jy, jz], axis=1).reshape(C, 1, 3)
        ddx = X - jx
        ddy = Y - jy
        ddz = Z - jz
        d2 = ddx * ddx + ddy * ddy + ddz * ddz
        return jnp.minimum(dist, d2)

    lax.fori_loop(1, m, body, dist0)


def _fps(pos_c, m):
    C, P, _ = pos_c.shape
    px = pos_c[:, :, 0]
    py = pos_c[:, :, 1]
    pz = pos_c[:, :, 2]
    return pl.pallas_call(
        functools.partial(_fps_body, m),
        out_shape=jax.ShapeDtypeStruct((C, m, 3), jnp.float32),
    )(px, py, pz)


def _radius_body(r2, n, sx_ref, sy_ref, sz_ref, dx_ref, dy_ref, dz_ref,
                 col_ref, cnt_ref):
    cid = pl.program_id(0)
    sx = sx_ref[0]
    sy = sy_ref[0]
    sz = sz_ref[0]
    dx = dx_ref[0]
    dy = dy_ref[0]
    dz = dz_ref[0]
    q = dx.shape[0]
    tri = (lax.broadcasted_iota(jnp.int32, (128, 128), 0)
           <= lax.broadcasted_iota(jnp.int32, (128, 128), 1)
           ).astype(jnp.bfloat16)
    pieces = []
    prefix = jnp.zeros((q, 1), jnp.float32)
    a = 0
    while a < n:
        w = min(128, n - a)
        ddx = dx - sx[:, a:a + w]
        ddy = dy - sy[:, a:a + w]
        ddz = dz - sz[:, a:a + w]
        d2 = ddx * ddx + ddy * ddy + ddz * ddz
        wint = (d2 <= r2).astype(jnp.bfloat16)
        local = jnp.dot(wint, tri[:w, :w],
                        preferred_element_type=jnp.float32)
        ci_t = local + prefix
        prefix = ci_t[:, w - 1:w]
        pieces.append(ci_t)
        a += w
    ci = jnp.concatenate(pieces, axis=1)
    cnt = prefix.astype(jnp.int32)
    cols = []
    for k in range(NN):
        cols.append(jnp.sum((ci <= jnp.float32(k)).astype(jnp.int32),
                            axis=1, keepdims=True))
    col = jnp.concatenate(cols, axis=1)
    col = jnp.minimum(col, n - 1) + cid * n
    col_ref[0] = col
    cnt_ref[0] = cnt


def _radius(pos_src_c, pos_dst_c, r2):
    C, n, _ = pos_src_c.shape
    q = pos_dst_c.shape[1]
    srcs = [pos_src_c[:, :, d].reshape(C, 1, n) for d in range(3)]
    dsts = [pos_dst_c[:, :, d:d + 1] for d in range(3)]
    grid = (C,)
    in_specs = (
        [pl.BlockSpec((1, 1, n), lambda c: (c, 0, 0)) for _ in range(3)]
        + [pl.BlockSpec((1, q, 1), lambda c: (c, 0, 0)) for _ in range(3)]
    )
    out_specs = [
        pl.BlockSpec((1, q, NN), lambda c: (c, 0, 0)),
        pl.BlockSpec((1, q, 1), lambda c: (c, 0, 0)),
    ]
    return pl.pallas_call(
        functools.partial(_radius_body, r2, n),
        grid=grid,
        in_specs=in_specs,
        out_specs=out_specs,
        out_shape=[
            jax.ShapeDtypeStruct((C, q, NN), jnp.int32),
            jax.ShapeDtypeStruct((C, q, 1), jnp.int32),
        ],
    )(*srcs, *dsts)


UNROLL = 2


def _sc_gather(table, idx2d):
    V, D = table.shape
    n_rows, CH = idx2d.shape
    B = n_rows * CH
    info = plsc.get_sparse_core_info()
    NC, NS = info.num_cores, info.num_subcores
    NW = NC * NS
    n_ch = n_rows // NW
    n_g = n_ch // UNROLL
    mesh = plsc.VectorSubcoreMesh(core_axis_name="c", subcore_axis_name="s")

    @functools.partial(
        pl.kernel,
        mesh=mesh,
        compiler_params=pltpu.CompilerParams(use_tc_tiling_on_sc=False),
        out_type=jax.ShapeDtypeStruct((B, D), jnp.float32),
        scratch_types=(
            [pltpu.VMEM((n_ch, CH), jnp.int32)]
            + [pltpu.VMEM((CH, D), jnp.float32) for _ in range(UNROLL)]
            + [pltpu.SemaphoreType.DMA for _ in range(2 * UNROLL)]
        ),
    )
    def k(table_hbm, idx_hbm, out_hbm, idx_v, *bufs_sems):
        bufs = bufs_sems[:UNROLL]
        gsems = bufs_sems[UNROLL:2 * UNROLL]
        osems = bufs_sems[2 * UNROLL:]
        wid = lax.axis_index("s") * NC + lax.axis_index("c")
        base = wid * n_ch
        pltpu.sync_copy(idx_hbm.at[pl.ds(base, n_ch)], idx_v)

        def body(g, carry):
            j0 = g * UNROLL
            cps = [
                pltpu.async_copy(table_hbm.at[idx_v.at[j0 + u]], bufs[u],
                                 gsems[u])
                for u in range(UNROLL)
            ]
            outs = []
            for u in range(UNROLL):
                cps[u].wait()
                outs.append(pltpu.async_copy(
                    bufs[u], out_hbm.at[pl.ds((base + j0 + u) * CH, CH)],
                    osems[u]))
            for o in outs:
                o.wait()
            return carry

        lax.fori_loop(0, n_g, body, 0)

    return k(table, idx2d)


def _conv_body(x_dim, rows_ref, dpos_ref, cnt_ref, w0_ref, b0_ref, w1_ref,
               b1_ref, w2_ref, b2_ref, out_ref):
    rows = rows_ref[...]
    dpos = dpos_ref[0]
    cnt = cnt_ref[0]
    Bq = dpos.shape[0]
    Be = rows.shape[0]
    drep = jnp.broadcast_to(dpos[:, None, :], (Bq, NN, 3)).reshape(Be, 3)
    rel = (rows[:, x_dim:x_dim + 3] - drep).astype(jnp.bfloat16)
    w0 = w0_ref[...].astype(jnp.bfloat16)
    if x_dim > 0:
        z = (jnp.dot(rows[:, :x_dim].astype(jnp.bfloat16), w0[:x_dim],
                     preferred_element_type=jnp.float32)
             + jnp.dot(rel, w0[x_dim:x_dim + 3],
                       preferred_element_type=jnp.float32)) + b0_ref[...]
    else:
        z = jnp.dot(rel, w0[:3], preferred_element_type=jnp.float32) \
            + b0_ref[...]
    h = jnp.maximum(z, 0.0).astype(jnp.bfloat16)
    z = jnp.dot(h, w1_ref[...].astype(jnp.bfloat16),
                preferred_element_type=jnp.float32) + b1_ref[...]
    h = jnp.maximum(z, 0.0).astype(jnp.bfloat16)
    z = jnp.dot(h, w2_ref[...].astype(jnp.bfloat16),
                preferred_element_type=jnp.float32) + b2_ref[...]
    k_e = lax.broadcasted_iota(jnp.int32, (Bq, NN, 1), 1).reshape(Be, 1)
    cnt_rep = jnp.broadcast_to(cnt[:, None, :], (Bq, NN, 1)).reshape(Be, 1)
    elig = k_e < cnt_rep
    zm = jnp.where(elig, z, -jnp.inf)
    y = jnp.max(zm.reshape(Bq, NN, z.shape[1]), axis=1)
    y = jnp.where(cnt > 0, y, 0.0)
    out_ref[0] = y


def _conv(rows, dst_pos_c, cnt, params, x_dim, bq):
    C, q, _ = dst_pos_c.shape
    (w0, b0), (w1, b1), (w2, b2) = params
    dout = w2.shape[1]
    dt = rows.shape[1]
    nb = q // bq
    be = bq * NN
    grid = (C, nb)
    in_specs = [
        pl.BlockSpec((be, dt), lambda c, b: (c * nb + b, 0)),
        pl.BlockSpec((1, bq, 3), lambda c, b: (c, b, 0)),
        pl.BlockSpec((1, bq, 1), lambda c, b: (c, b, 0)),
    ]
    for wgt, bias in ((w0, b0), (w1, b1), (w2, b2)):
        in_specs.append(
            pl.BlockSpec(wgt.shape, lambda c, b: (0, 0)))
        in_specs.append(
            pl.BlockSpec((1, bias.shape[0]), lambda c, b: (0, 0)))
    return pl.pallas_call(
        functools.partial(_conv_body, x_dim),
        grid=grid,
        in_specs=in_specs,
        out_specs=pl.BlockSpec((1, bq, dout), lambda c, b: (c, b, 0)),
        out_shape=jax.ShapeDtypeStruct((C, q, dout), jnp.float32),
    )(rows, dst_pos_c, cnt,
      w0, b0.reshape(1, -1), w1, b1.reshape(1, -1), w2, b2.reshape(1, -1))


def _sa3_body(x_ref, p_ref, w0_ref, b0_ref, w1_ref, b1_ref, w2_ref, b2_ref,
              out_ref):
    x = x_ref[0].astype(jnp.bfloat16)
    p = p_ref[0].astype(jnp.bfloat16)
    w0 = w0_ref[...].astype(jnp.bfloat16)
    z = (jnp.dot(x, w0[:256], preferred_element_type=jnp.float32)
         + jnp.dot(p, w0[256:259], preferred_element_type=jnp.float32)) \
        + b0_ref[...]
    h = jnp.maximum(z, 0.0).astype(jnp.bfloat16)
    z = jnp.dot(h, w1_ref[...].astype(jnp.bfloat16),
                preferred_element_type=jnp.float32) + b1_ref[...]
    h = jnp.maximum(z, 0.0).astype(jnp.bfloat16)
    z = jnp.dot(h, w2_ref[...].astype(jnp.bfloat16),
                preferred_element_type=jnp.float32) + b2_ref[...]
    out_ref[0, 0, :] = jnp.max(z, axis=0)


def _sa3(x2_c, pos2_c, params):
    C, m, _ = x2_c.shape
    (w0, b0), (w1, b1), (w2, b2) = params
    enc = w2.shape[1]
    in_specs = [
        pl.BlockSpec((1, m, 256), lambda c: (c, 0, 0)),
        pl.BlockSpec((1, m, 3), lambda c: (c, 0, 0)),
    ]
    for wgt, bias in ((w0, b0), (w1, b1), (w2, b2)):
        in_specs.append(pl.BlockSpec(wgt.shape, lambda c: (0, 0)))
        in_specs.append(pl.BlockSpec((1, bias.shape[0]), lambda c: (0, 0)))
    out = pl.pallas_call(
        _sa3_body,
        grid=(C,),
        in_specs=in_specs,
        out_specs=pl.BlockSpec((1, 1, enc), lambda c: (c, 0, 0)),
        out_shape=jax.ShapeDtypeStruct((C, 1, enc), jnp.float32),
    )(x2_c, pos2_c,
      w0, b0.reshape(1, -1), w1, b1.reshape(1, -1), w2, b2.reshape(1, -1))
    return out.reshape(C, enc)


def kernel(pos, ptr, mlp1, mlp2, mlp3):
    C = ptr.shape[0] - 1
    P = pos.shape[0] // C
    pos_c = pos.reshape(C, P, 3)

    pos1_c = _fps(pos_c, M1)
    pos1p_c = jnp.concatenate(
        [pos1_c, jnp.full((C, M1P - M1, 3), 1e6, jnp.float32)], axis=1)
    col1, cnt1 = _radius(pos_c, pos1p_c, R2_1)
    e1 = C * M1P * NN
    idx1 = col1.reshape(e1 // 256, 256)
    table1 = jnp.concatenate(
        [pos, jnp.zeros((C * P, 13), jnp.float32)], axis=1)
    rows1 = _sc_gather(table1, idx1)
    x1_c = _conv(rows1, pos1p_c, cnt1, mlp1, 0, 128)

    pos2_c = _fps(pos1_c, M2)
    pos2p_c = jnp.concatenate(
        [pos2_c, jnp.full((C, M2P - M2, 3), 1e6, jnp.float32)], axis=1)
    col2, cnt2 = _radius(pos1_c, pos2p_c, R2_2)
    e2 = C * M2P * NN
    idx2 = col2.reshape(e2 // 256, 256)
    table2 = jnp.concatenate(
        [x1_c[:, :M1].reshape(C * M1, 128), pos1_c.reshape(C * M1, 3),
         jnp.zeros((C * M1, 13), jnp.float32)], axis=1)
    rows2 = _sc_gather(table2, idx2)
    x2_c = _conv(rows2, pos2p_c, cnt2, mlp2, 128, 160)

    return _sa3(x2_c[:, :M2], pos2_c, mlp3)
